# Initial kernel scaffold; baseline (speedup 1.0000x reference)
import jax
import jax.numpy as jnp
from jax.experimental import pallas as pl


def kernel(tokens, token_embed, pos_embed, mask):
    raise NotImplementedError("write your pallas kernel here")


# speedup vs baseline: 1.8307x; 1.8307x over previous
import functools

import jax
import jax.numpy as jnp
from jax.experimental import pallas as pl
from jax.experimental.pallas import tpu as pltpu


def _fused_embed_kernel(tok_ref, emb_hbm, pos_ref, scale_ref, out_ref,
                        gbuf, sems, *, rows, tiles_per_core, use_scale):
    c = pl.program_id(0)
    t = pl.program_id(1)
    tile = c * tiles_per_core + t

    def issue(tile_idx, slot):
        base = tile_idx * rows
        for i in range(rows):
            pltpu.make_async_copy(emb_hbm.at[tok_ref[base + i]],
                                  gbuf.at[slot, i], sems.at[slot]).start()

    @pl.when(t == 0)
    def _cold():
        issue(tile, 0)

    @pl.when(t + 1 < tiles_per_core)
    def _prefetch():
        issue(tile + 1, (t + 1) % 2)

    slot = t % 2
    pltpu.make_async_copy(emb_hbm.at[pl.ds(0, rows)], gbuf.at[slot],
                          sems.at[slot]).wait()

    x = gbuf[slot] + pos_ref[...]
    if use_scale:
        x = x * scale_ref[0]
    out_ref[0] = x


def kernel(tokens, token_embed, pos_embed, mask):
    B, S = tokens.shape
    V, D = token_embed.shape
    out_dtype = jnp.promote_types(token_embed.dtype, pos_embed.dtype)

    tok = tokens.reshape(B * S)
    pos = pos_embed[0, :S, :].astype(out_dtype)

    if B % 2 == 0:
        n_cores, tiles_per_core = 2, B // 2
    else:
        n_cores, tiles_per_core = 1, B

    tpc = tiles_per_core

    in_specs = [
        pl.BlockSpec(memory_space=pl.ANY),
        pl.BlockSpec((S, D), lambda c, t, tok: (0, 0)),
    ]
    args = [token_embed, pos]
    use_scale = mask is not None
    if use_scale:
        if mask.ndim == 3:
            scale = jnp.sum(jax.nn.sigmoid(mask.astype(jnp.float32)),
                            axis=-1, keepdims=True)
        else:
            scale = mask.astype(jnp.float32).reshape(B, S, 1)
        in_specs.append(
            pl.BlockSpec((1, S, 1), lambda c, t, tok: (c * tpc + t, 0, 0)))
        args.append(scale)

    fn = functools.partial(_fused_embed_kernel, rows=S, tiles_per_core=tpc,
                          use_scale=use_scale)
    if not use_scale:
        def fn_plain(tok_ref, emb_hbm, pos_ref, out_ref, gbuf, sems):
            _fused_embed_kernel(tok_ref, emb_hbm, pos_ref, None, out_ref,
                                gbuf, sems, rows=S, tiles_per_core=tpc,
                                use_scale=False)
        fn = fn_plain

    out = pl.pallas_call(
        fn,
        out_shape=jax.ShapeDtypeStruct((B, S, D), out_dtype),
        grid_spec=pltpu.PrefetchScalarGridSpec(
            num_scalar_prefetch=1,
            grid=(n_cores, tiles_per_core),
            in_specs=in_specs,
            out_specs=pl.BlockSpec((1, S, D),
                                   lambda c, t, tok: (c * tpc + t, 0, 0)),
            scratch_shapes=[pltpu.VMEM((2, S, D), token_embed.dtype),
                            pltpu.SemaphoreType.DMA((2,))],
        ),
        compiler_params=pltpu.CompilerParams(
            dimension_semantics=("parallel", "arbitrary"),
            disable_bounds_checks=True,
            vmem_limit_bytes=32 * 1024 * 1024,
        ),
    )(tok, *args)
    return out
